# baseline (speedup 1.0000x reference)
import functools

import jax
import jax.numpy as jnp
from jax import lax
from jax.experimental import pallas as pl
from jax.experimental.pallas import tpu as pltpu
from jax.experimental.pallas import tpu_sc as plsc

B = 1024
F = 128
NP = 100000
NCQ = 5000
SCALE = 30.0
ALPHA = 128.0
LOG2E = 1.4426950408889634
LN2 = 0.6931471805599453

STATS_BLK = 20000
FLASH_BLK = 5000
K_STATS = NP // STATS_BLK
K_FLASH = NP // FLASH_BLK
NSTEPS = K_STATS + K_FLASH + 1

NUM_SC_WORKERS = 32
ROWS_PER_WORKER = B // NUM_SC_WORKERS


def _flash_step(tbl16, s8, m_ref, s_ref):
    n = tbl16.shape[0]
    dots = lax.dot_general(
        tbl16.astype(jnp.float8_e4m3fn), s8,
        (((1,), (1,)), ((), ())),
        preferred_element_type=jnp.float32).astype(jnp.bfloat16)
    bm = jnp.max(dots, axis=0, keepdims=True).astype(jnp.float32)
    m_new = jnp.maximum(m_ref[...], bm)
    e16 = jnp.exp2((dots - m_new.astype(jnp.bfloat16)) *
                   jnp.bfloat16(LOG2E / ALPHA))
    blksum = lax.dot_general(
        jnp.ones((1, n), jnp.bfloat16), e16,
        (((1,), (0,)), ((), ())), preferred_element_type=jnp.float32)
    s_ref[...] = s_ref[...] * jnp.exp(
        (m_ref[...] - m_new) * (1.0 / ALPHA)) + blksum
    m_ref[...] = m_new


def _flash_pair(tbl, s8, ma, sa, mb, sb):
    n = tbl.shape[0]
    h = ((n // 2) // 8) * 8
    _flash_step(tbl[:h], s8, ma, sa)
    _flash_step(tbl[h:], s8, mb, sb)


def _merge_streams(ma, sa, mb, sb):
    m = jnp.maximum(ma, mb)
    s = (sa * jnp.exp((ma - m) * (1.0 / ALPHA)) +
         sb * jnp.exp((mb - m) * (1.0 / ALPHA)))
    return m, s


def _fused_body(lut_s_ref, lut_f_ref, cq_ref, inp_ref,
                m1_out, s1_out, m2_out, s2_out, nb1_out, nb2_out, sl_out,
                ssq, nbad, s8l, s8q, ma, sa, mb, sb):
    i = pl.program_id(0)
    dims_row = (((1,), (0,)), ((), ()))
    dims_col = (((1,), (1,)), ((), ()))

    @pl.when(i == 0)
    def _init():
        ssq[...] = jnp.zeros_like(ssq)
        nbad[...] = jnp.zeros_like(nbad)

    @pl.when(i < K_STATS)
    def _stats():
        b16 = lut_s_ref[...].astype(jnp.bfloat16)
        ssq[...] += lax.dot_general(
            jnp.ones((1, STATS_BLK), jnp.bfloat16), b16 * b16, dims_row,
            preferred_element_type=jnp.float32)
        row_abs = lax.dot_general(
            jnp.ones((1, F), jnp.bfloat16), jnp.abs(b16), dims_col,
            preferred_element_type=jnp.float32)
        nbad[...] += jnp.sum(
            jnp.where(row_abs == 0.0, 1.0, 0.0), keepdims=True)

    @pl.when(i == K_STATS - 1)
    def _mid():
        inv_lut = 1.0 / jnp.maximum(jnp.sqrt(ssq[...]), 1e-12)
        cqb = cq_ref[...]
        cq_ssq = jnp.sum(cqb * cqb, axis=0, keepdims=True)
        inv_cq = 1.0 / jnp.maximum(jnp.sqrt(cq_ssq), 1e-12)
        cq_row_abs = lax.dot_general(
            jnp.ones((1, F), jnp.bfloat16),
            jnp.abs(cqb.astype(jnp.bfloat16)), dims_col,
            preferred_element_type=jnp.float32)
        nb2_out[...] = jnp.sum(
            jnp.where(cq_row_abs == 0.0, 1.0, 0.0), keepdims=True)
        nb1_out[...] = nbad[...]
        x = inp_ref[...]
        rn = jnp.sqrt(jnp.sum(x * x, axis=1, keepdims=True))
        ninp = (SCALE * x) / jnp.maximum(rn, 1e-12)
        sl = ninp * inv_lut
        sl_out[...] = sl
        s8l[...] = (sl * ALPHA).astype(jnp.float8_e4m3fn)
        s8q[...] = (ninp * inv_cq * ALPHA).astype(jnp.float8_e4m3fn)
        ma[...] = jnp.full_like(ma, -1e30)
        sa[...] = jnp.zeros_like(sa)
        mb[...] = jnp.full_like(mb, -1e30)
        sb[...] = jnp.zeros_like(sb)

    @pl.when(jnp.logical_and(i >= K_STATS, i < K_STATS + K_FLASH))
    def _flash_lut():
        _flash_pair(lut_f_ref[...], s8l[...], ma, sa, mb, sb)

    @pl.when(i == NSTEPS - 1)
    def _flash_cq():
        m1v, s1v = _merge_streams(ma[...], sa[...], mb[...], sb[...])
        m1_out[...] = m1v
        s1_out[...] = s1v
        ma[...] = jnp.full_like(ma, -1e30)
        sa[...] = jnp.zeros_like(sa)
        mb[...] = jnp.full_like(mb, -1e30)
        sb[...] = jnp.zeros_like(sb)
        _flash_pair(cq_ref[...], s8q[...], ma, sa, mb, sb)
        m2v, s2v = _merge_streams(ma[...], sa[...], mb[...], sb[...])
        m2_out[...] = m2v
        s2_out[...] = s2v


def _finish_body(m1_ref, s1_ref, m2_ref, s2_ref, nb1_ref, nb2_ref, sl_ref,
                 g_ref, lab_ref, out_ref):
    g = g_ref[...]
    ones = jnp.ones((1, F), jnp.float32)
    dims = (((1,), (1,)), ((), ()))
    zdot = lax.dot_general(ones, sl_ref[...] * g, dims,
                           preferred_element_type=jnp.float32)
    gabs = lax.dot_general(ones, jnp.abs(g), dims,
                           preferred_element_type=jnp.float32)
    badrow = gabs == 0.0
    lab = lab_ref[...]
    valid = lab < NP
    corr = jnp.logical_and(valid, badrow)
    c = 1.0 / ALPHA
    m1, s1 = m1_ref[...] * c, s1_ref[...]
    m2, s2 = m2_ref[...] * c, s2_ref[...]
    nb1, nb2 = nb1_ref[...], nb2_ref[...]
    s1 = s1 + jnp.where(
        nb1 > 0.0, nb1 * (jnp.exp(-SCALE - m1) - jnp.exp(-m1)), 0.0)
    s2 = s2 + jnp.where(
        nb2 > 0.0, nb2 * (jnp.exp(-SCALE - m2) - jnp.exp(-m2)), 0.0)
    m = jnp.maximum(m1, m2)
    s = s1 * jnp.exp(m1 - m) + s2 * jnp.exp(m2 - m)
    mc = jnp.where(corr, jnp.maximum(m, SCALE), m)
    s_fix = s * jnp.exp(m - mc) + jnp.where(
        corr, jnp.exp(SCALE - mc) - jnp.exp(-SCALE - mc), 0.0)
    v = jnp.where(corr, SCALE, zdot)
    nll = mc + jnp.log(s_fix) - v
    nll = jnp.where(valid, nll, 0.0)
    out_ref[...] = jnp.sum(nll, keepdims=True) * (1.0 / B)


@functools.cache
def _make_sc_gather():
    @functools.partial(
        pl.kernel,
        out_type=jax.ShapeDtypeStruct((B, F), jnp.float32),
        mesh=plsc.VectorSubcoreMesh(core_axis_name="c", subcore_axis_name="s"),
        scratch_types=[
            pltpu.VMEM((ROWS_PER_WORKER,), jnp.int32),
            pltpu.VMEM((ROWS_PER_WORKER, F), jnp.float32),
            pltpu.SemaphoreType.DMA,
        ],
    )
    def _sc_gather(table_hbm, idx_hbm, out_hbm, idx_v, rows_v, sem):
        wid = lax.axis_index("s") * 2 + lax.axis_index("c")
        base = wid * ROWS_PER_WORKER
        pltpu.sync_copy(idx_hbm.at[pl.ds(base, ROWS_PER_WORKER)], idx_v)
        pltpu.async_copy(table_hbm.at[idx_v], rows_v, sem).wait()
        pltpu.sync_copy(rows_v, out_hbm.at[pl.ds(base, ROWS_PER_WORKER)])

    return _sc_gather


def _fused_call(lut, cq, inputs, interpret=False):
    one = jax.ShapeDtypeStruct((1, 1), jnp.float32)
    row = jax.ShapeDtypeStruct((1, B), jnp.float32)
    return pl.pallas_call(
        _fused_body,
        grid=(NSTEPS,),
        in_specs=[
            pl.BlockSpec((STATS_BLK, F),
                         lambda i: (jnp.minimum(i, K_STATS - 1), 0)),
            pl.BlockSpec((FLASH_BLK, F),
                         lambda i: (jnp.clip(i - K_STATS, 0, K_FLASH - 1),
                                    0)),
            pl.BlockSpec((NCQ, F), lambda i: (0, 0)),
            pl.BlockSpec((B, F), lambda i: (0, 0)),
        ],
        out_specs=[
            pl.BlockSpec((1, B), lambda i: (0, 0)),
            pl.BlockSpec((1, B), lambda i: (0, 0)),
            pl.BlockSpec((1, B), lambda i: (0, 0)),
            pl.BlockSpec((1, B), lambda i: (0, 0)),
            pl.BlockSpec((1, 1), lambda i: (0, 0)),
            pl.BlockSpec((1, 1), lambda i: (0, 0)),
            pl.BlockSpec((B, F), lambda i: (0, 0)),
        ],
        out_shape=[row, row, row, row, one, one,
                   jax.ShapeDtypeStruct((B, F), jnp.float32)],
        scratch_shapes=[
            pltpu.VMEM((1, F), jnp.float32),
            pltpu.VMEM((1, 1), jnp.float32),
            pltpu.VMEM((B, F), jnp.float8_e4m3fn),
            pltpu.VMEM((B, F), jnp.float8_e4m3fn),
            pltpu.VMEM((1, B), jnp.float32),
            pltpu.VMEM((1, B), jnp.float32),
            pltpu.VMEM((1, B), jnp.float32),
            pltpu.VMEM((1, B), jnp.float32),
        ],
        compiler_params=pltpu.CompilerParams(
            dimension_semantics=("arbitrary",)),
        interpret=interpret,
    )(lut, lut, cq, inputs)


def _finish_call(m1, s1, m2, s2, nb1, nb2, scaled_lut, gathered, lab2d,
                 interpret=False):
    return pl.pallas_call(
        _finish_body,
        out_shape=jax.ShapeDtypeStruct((1, 1), jnp.float32),
        interpret=interpret,
    )(m1, s1, m2, s2, nb1, nb2, scaled_lut, gathered, lab2d)


def kernel(inputs, label, lut, cq):
    label = label.astype(jnp.int32)
    clip = jnp.clip(label, 0, NP - 1)
    gathered = _make_sc_gather()(lut, clip)
    m1, s1, m2, s2, nb1, nb2, scaled_lut = _fused_call(lut, cq, inputs)
    out = _finish_call(m1, s1, m2, s2, nb1, nb2, scaled_lut, gathered,
                       label.reshape(1, B))
    return out[0, 0]


# speedup vs baseline: 1.0380x; 1.0380x over previous
import functools

import jax
import jax.numpy as jnp
from jax import lax
from jax.experimental import pallas as pl
from jax.experimental.pallas import tpu as pltpu
from jax.experimental.pallas import tpu_sc as plsc

B = 1024
F = 128
NP = 100000
NCQ = 5000
SCALE = 30.0
ALPHA = 128.0
LOG2E = 1.4426950408889634
LN2 = 0.6931471805599453

STATS_BLK = 20000
FLASH_BLK = 5000
K_STATS = NP // STATS_BLK
K_FLASH = NP // FLASH_BLK
NSTEPS = K_STATS + K_FLASH + 1

NUM_SC_WORKERS = 32
ROWS_PER_WORKER = B // NUM_SC_WORKERS


def _flash_step(tbl16, s8, m_ref, s_ref):
    n = tbl16.shape[0]
    dots = lax.dot_general(
        tbl16.astype(jnp.float8_e4m3fn), s8,
        (((1,), (1,)), ((), ())),
        preferred_element_type=jnp.float32).astype(jnp.bfloat16)
    bm = jnp.max(dots, axis=0, keepdims=True).astype(jnp.float32)
    m_new = jnp.maximum(m_ref[...], bm)
    e16 = jnp.exp2((dots - m_new.astype(jnp.bfloat16)) *
                   jnp.bfloat16(LOG2E / ALPHA))
    blksum = lax.dot_general(
        jnp.ones((1, n), jnp.bfloat16), e16,
        (((1,), (0,)), ((), ())), preferred_element_type=jnp.float32)
    s_ref[...] = s_ref[...] * jnp.exp(
        (m_ref[...] - m_new) * (1.0 / ALPHA)) + blksum
    m_ref[...] = m_new


def _fused_body(lut_s_ref, lut_f_ref, cq_ref, inp_ref,
                m1_out, s1_out, m2_out, s2_out, nb1_out, nb2_out, sl_out,
                ssq, nbad, s8l, s8q, m1, s1):
    i = pl.program_id(0)
    dims_row = (((1,), (0,)), ((), ()))
    dims_col = (((1,), (1,)), ((), ()))

    @pl.when(i == 0)
    def _init():
        ssq[...] = jnp.zeros_like(ssq)
        nbad[...] = jnp.zeros_like(nbad)

    @pl.when(i < K_STATS)
    def _stats():
        b16 = lut_s_ref[...].astype(jnp.bfloat16)
        ssq[...] += lax.dot_general(
            jnp.ones((1, STATS_BLK), jnp.bfloat16), b16 * b16, dims_row,
            preferred_element_type=jnp.float32)
        row_abs = lax.dot_general(
            jnp.ones((1, F), jnp.bfloat16), jnp.abs(b16), dims_col,
            preferred_element_type=jnp.float32)
        nbad[...] += jnp.sum(
            jnp.where(row_abs == 0.0, 1.0, 0.0), keepdims=True)

    @pl.when(i == K_STATS - 1)
    def _mid():
        inv_lut = 1.0 / jnp.maximum(jnp.sqrt(ssq[...]), 1e-12)
        cqb = cq_ref[...]
        cq_ssq = jnp.sum(cqb * cqb, axis=0, keepdims=True)
        inv_cq = 1.0 / jnp.maximum(jnp.sqrt(cq_ssq), 1e-12)
        cq_row_abs = lax.dot_general(
            jnp.ones((1, F), jnp.bfloat16),
            jnp.abs(cqb.astype(jnp.bfloat16)), dims_col,
            preferred_element_type=jnp.float32)
        nb2_out[...] = jnp.sum(
            jnp.where(cq_row_abs == 0.0, 1.0, 0.0), keepdims=True)
        nb1_out[...] = nbad[...]
        x = inp_ref[...]
        rn = jnp.sqrt(jnp.sum(x * x, axis=1, keepdims=True))
        ninp = (SCALE * x) / jnp.maximum(rn, 1e-12)
        sl = ninp * inv_lut
        sl_out[...] = sl
        s8l[...] = (sl * ALPHA).astype(jnp.float8_e4m3fn)
        s8q[...] = (ninp * inv_cq * ALPHA).astype(jnp.float8_e4m3fn)
        m1[...] = jnp.full_like(m1, -1e30)
        s1[...] = jnp.zeros_like(s1)

    @pl.when(jnp.logical_and(i >= K_STATS, i < K_STATS + K_FLASH))
    def _flash_lut():
        _flash_step(lut_f_ref[...], s8l[...], m1, s1)

    @pl.when(i == NSTEPS - 1)
    def _flash_cq():
        m1_out[...] = m1[...]
        s1_out[...] = s1[...]
        m1[...] = jnp.full_like(m1, -1e30)
        s1[...] = jnp.zeros_like(s1)
        _flash_step(cq_ref[...], s8q[...], m1, s1)
        m2_out[...] = m1[...]
        s2_out[...] = s1[...]


def _finish_body(m1_ref, s1_ref, m2_ref, s2_ref, nb1_ref, nb2_ref, sl_ref,
                 g_ref, lab_ref, out_ref):
    g = g_ref[...]
    ones = jnp.ones((1, F), jnp.float32)
    dims = (((1,), (1,)), ((), ()))
    zdot = lax.dot_general(ones, sl_ref[...] * g, dims,
                           preferred_element_type=jnp.float32)
    gabs = lax.dot_general(ones, jnp.abs(g), dims,
                           preferred_element_type=jnp.float32)
    badrow = gabs == 0.0
    lab = lab_ref[...]
    valid = lab < NP
    corr = jnp.logical_and(valid, badrow)
    c = 1.0 / ALPHA
    m1, s1 = m1_ref[...] * c, s1_ref[...]
    m2, s2 = m2_ref[...] * c, s2_ref[...]
    nb1, nb2 = nb1_ref[...], nb2_ref[...]
    s1 = s1 + jnp.where(
        nb1 > 0.0, nb1 * (jnp.exp(-SCALE - m1) - jnp.exp(-m1)), 0.0)
    s2 = s2 + jnp.where(
        nb2 > 0.0, nb2 * (jnp.exp(-SCALE - m2) - jnp.exp(-m2)), 0.0)
    m = jnp.maximum(m1, m2)
    s = s1 * jnp.exp(m1 - m) + s2 * jnp.exp(m2 - m)
    mc = jnp.where(corr, jnp.maximum(m, SCALE), m)
    s_fix = s * jnp.exp(m - mc) + jnp.where(
        corr, jnp.exp(SCALE - mc) - jnp.exp(-SCALE - mc), 0.0)
    v = jnp.where(corr, SCALE, zdot)
    nll = mc + jnp.log(s_fix) - v
    nll = jnp.where(valid, nll, 0.0)
    out_ref[...] = jnp.sum(nll, keepdims=True) * (1.0 / B)


@functools.cache
def _make_sc_gather():
    @functools.partial(
        pl.kernel,
        out_type=jax.ShapeDtypeStruct((B, F), jnp.float32),
        mesh=plsc.VectorSubcoreMesh(core_axis_name="c", subcore_axis_name="s"),
        scratch_types=[
            pltpu.VMEM((ROWS_PER_WORKER,), jnp.int32),
            pltpu.VMEM((ROWS_PER_WORKER, F), jnp.float32),
            pltpu.SemaphoreType.DMA,
        ],
    )
    def _sc_gather(table_hbm, idx_hbm, out_hbm, idx_v, rows_v, sem):
        wid = lax.axis_index("s") * 2 + lax.axis_index("c")
        base = wid * ROWS_PER_WORKER
        pltpu.sync_copy(idx_hbm.at[pl.ds(base, ROWS_PER_WORKER)], idx_v)
        pltpu.async_copy(table_hbm.at[idx_v], rows_v, sem).wait()
        pltpu.sync_copy(rows_v, out_hbm.at[pl.ds(base, ROWS_PER_WORKER)])

    return _sc_gather


def _fused_call(lut, cq, inputs, interpret=False):
    one = jax.ShapeDtypeStruct((1, 1), jnp.float32)
    row = jax.ShapeDtypeStruct((1, B), jnp.float32)
    return pl.pallas_call(
        _fused_body,
        grid=(NSTEPS,),
        in_specs=[
            pl.BlockSpec((STATS_BLK, F),
                         lambda i: (jnp.minimum(i, K_STATS - 1), 0)),
            pl.BlockSpec((FLASH_BLK, F),
                         lambda i: (jnp.clip(i - K_STATS, 0, K_FLASH - 1),
                                    0)),
            pl.BlockSpec((NCQ, F), lambda i: (0, 0)),
            pl.BlockSpec((B, F), lambda i: (0, 0)),
        ],
        out_specs=[
            pl.BlockSpec((1, B), lambda i: (0, 0)),
            pl.BlockSpec((1, B), lambda i: (0, 0)),
            pl.BlockSpec((1, B), lambda i: (0, 0)),
            pl.BlockSpec((1, B), lambda i: (0, 0)),
            pl.BlockSpec((1, 1), lambda i: (0, 0)),
            pl.BlockSpec((1, 1), lambda i: (0, 0)),
            pl.BlockSpec((B, F), lambda i: (0, 0)),
        ],
        out_shape=[row, row, row, row, one, one,
                   jax.ShapeDtypeStruct((B, F), jnp.float32)],
        scratch_shapes=[
            pltpu.VMEM((1, F), jnp.float32),
            pltpu.VMEM((1, 1), jnp.float32),
            pltpu.VMEM((B, F), jnp.float8_e4m3fn),
            pltpu.VMEM((B, F), jnp.float8_e4m3fn),
            pltpu.VMEM((1, B), jnp.float32),
            pltpu.VMEM((1, B), jnp.float32),
        ],
        compiler_params=pltpu.CompilerParams(
            dimension_semantics=("arbitrary",)),
        interpret=interpret,
    )(lut, lut, cq, inputs)


def _finish_call(m1, s1, m2, s2, nb1, nb2, scaled_lut, gathered, lab2d,
                 interpret=False):
    return pl.pallas_call(
        _finish_body,
        out_shape=jax.ShapeDtypeStruct((1, 1), jnp.float32),
        interpret=interpret,
    )(m1, s1, m2, s2, nb1, nb2, scaled_lut, gathered, lab2d)


def kernel(inputs, label, lut, cq):
    label = label.astype(jnp.int32)
    clip = jnp.clip(label, 0, NP - 1)
    gathered = _make_sc_gather()(lut, clip)
    m1, s1, m2, s2, nb1, nb2, scaled_lut = _fused_call(lut, cq, inputs)
    out = _finish_call(m1, s1, m2, s2, nb1, nb2, scaled_lut, gathered,
                       label.reshape(1, B))
    return out[0, 0]
